# baseline (speedup 1.0000x reference)
import functools

import jax
import jax.numpy as jnp
from jax import lax
from jax.experimental import pallas as pl
from jax.experimental.pallas import tpu as pltpu
from jax.experimental.pallas import tpu_sc as plsc

N_OUT = 64
LANES = 16
N_SUB = 16
BATCH_BLOCK = 1024


def _build_w_sc(src, dst, w, zeros_tab, n_in, n_rec):
    E = src.shape[0]
    w_size = n_in * N_OUT
    base = n_rec - N_OUT
    ept = E // N_SUB
    n_chunks = ept // LANES
    n_rows = ept // 128
    z_len = w_size // N_SUB

    mesh = plsc.VectorSubcoreMesh(core_axis_name="c", subcore_axis_name="s",
                                  num_cores=1)

    @functools.partial(
        pl.kernel,
        mesh=mesh,
        out_type=jax.ShapeDtypeStruct((w_size,), jnp.float32),
        compiler_params=pltpu.CompilerParams(needs_layout_passes=False),
        scratch_types=[
            pltpu.VMEM((ept,), jnp.int32),
            pltpu.VMEM((ept,), jnp.int32),
            pltpu.VMEM((ept,), jnp.float32),
            pltpu.VMEM((n_rows, 128), jnp.int32),
            pltpu.VMEM((n_rows, 128), jnp.float32),
            pltpu.VMEM_SHARED((w_size,), jnp.float32),
            pltpu.SemaphoreType.DMA,
            pltpu.SemaphoreType.DMA,
            pltpu.SemaphoreType.DMA,
        ],
    )
    def build_w(src_hbm, dst_hbm, w_hbm, zeros_hbm, out_hbm,
                src_v, dst_v, w_v, idx_v, val_v, w_sh,
                sem_in, sem_z, sem_a):
        cid = lax.axis_index("c")
        sid = lax.axis_index("s")

        @pl.when(cid == 0)
        def _():
            base_e = sid * ept

            cp_s = pltpu.async_copy(src_hbm.at[pl.ds(base_e, ept)], src_v,
                                    sem_in)
            cp_d = pltpu.async_copy(dst_hbm.at[pl.ds(base_e, ept)], dst_v,
                                    sem_in)
            cp_w = pltpu.async_copy(w_hbm.at[pl.ds(base_e, ept)], w_v,
                                    sem_in)
            cp_z = pltpu.async_copy(zeros_hbm.at[pl.ds(sid * z_len, z_len)],
                                    w_sh.at[pl.ds(sid * z_len, z_len)],
                                    sem_z)
            cp_s.wait()
            cp_d.wait()
            cp_w.wait()

            for c in range(n_chunks):
                sl = pl.ds(c * LANES, LANES)
                s16 = src_v[sl]
                d16 = dst_v[sl]
                w16 = w_v[sl]
                m = (s16 < n_in) & (d16 >= base)
                idx = jnp.where(m, s16 * N_OUT + (d16 - base), 0)
                val = jnp.where(m, w16, 0.0)
                out_sl = pl.ds((c % 8) * LANES, LANES)
                idx_v[c // 8, out_sl] = idx
                val_v[c // 8, out_sl] = val

            cp_z.wait()
            plsc.subcore_barrier()

            adds = [
                pltpu.async_copy(val_v.at[j], w_sh.at[idx_v.at[j]], sem_a,
                                 add=True)
                for j in range(n_rows)
            ]
            for cp in adds:
                cp.wait()

            plsc.subcore_barrier()

            pltpu.sync_copy(w_sh.at[pl.ds(sid * z_len, z_len)],
                            out_hbm.at[pl.ds(sid * z_len, z_len)])

    return build_w(src, dst, w, zeros_tab)


def _matmul_tc(obs, w_table, bias):
    batch, n_in = obs.shape
    n_rec = bias.shape[0]
    base = n_rec - N_OUT
    n_blocks = batch // BATCH_BLOCK

    def body(obs_ref, w_ref, b_ref, out_ref):
        acc = jnp.dot(obs_ref[...], w_ref[...],
                      preferred_element_type=jnp.float32)
        b = b_ref[0, base:base + N_OUT]
        out_ref[...] = jnp.tanh(acc + b[None, :])

    return pl.pallas_call(
        body,
        grid=(n_blocks,),
        in_specs=[
            pl.BlockSpec((BATCH_BLOCK, n_in), lambda i: (i, 0)),
            pl.BlockSpec((n_in, N_OUT), lambda i: (0, 0)),
            pl.BlockSpec((1, n_rec), lambda i: (0, 0)),
        ],
        out_specs=pl.BlockSpec((BATCH_BLOCK, N_OUT), lambda i: (i, 0)),
        out_shape=jax.ShapeDtypeStruct((batch, N_OUT), jnp.float32),
    )(obs, w_table, bias.reshape(1, n_rec))


def kernel(obs, src, dst, w, bias):
    n_in = obs.shape[1]
    n_rec = bias.shape[0]
    src = src.astype(jnp.int32)
    dst = dst.astype(jnp.int32)
    zeros_tab = jnp.zeros((n_in * N_OUT,), jnp.float32)
    w_flat = _build_w_sc(src, dst, w, zeros_tab, n_in, n_rec)
    return _matmul_tc(obs, w_flat.reshape(n_in, N_OUT), bias)


# speedup vs baseline: 1.0332x; 1.0332x over previous
import functools

import jax
import jax.numpy as jnp
from jax import lax
from jax.experimental import pallas as pl
from jax.experimental.pallas import tpu as pltpu
from jax.experimental.pallas import tpu_sc as plsc

N_OUT = 64
LANES = 16
N_SUB = 16
BATCH_BLOCK = 2048


def _build_w_sc(src, dst, w, zeros_tab, n_in, n_rec):
    E = src.shape[0]
    w_size = n_in * N_OUT
    base = n_rec - N_OUT
    ept = E // N_SUB
    n_chunks = ept // LANES
    n_rows = ept // 128
    z_len = w_size // N_SUB

    mesh = plsc.VectorSubcoreMesh(core_axis_name="c", subcore_axis_name="s",
                                  num_cores=1)

    @functools.partial(
        pl.kernel,
        mesh=mesh,
        out_type=jax.ShapeDtypeStruct((w_size,), jnp.float32),
        compiler_params=pltpu.CompilerParams(needs_layout_passes=False),
        scratch_types=[
            pltpu.VMEM((ept,), jnp.int32),
            pltpu.VMEM((ept,), jnp.int32),
            pltpu.VMEM((ept,), jnp.float32),
            pltpu.VMEM((n_rows, 128), jnp.int32),
            pltpu.VMEM((n_rows, 128), jnp.float32),
            pltpu.VMEM_SHARED((w_size,), jnp.float32),
            pltpu.SemaphoreType.DMA,
            pltpu.SemaphoreType.DMA,
            pltpu.SemaphoreType.DMA,
        ],
    )
    def build_w(src_hbm, dst_hbm, w_hbm, zeros_hbm, out_hbm,
                src_v, dst_v, w_v, idx_v, val_v, w_sh,
                sem_in, sem_z, sem_a):
        cid = lax.axis_index("c")
        sid = lax.axis_index("s")

        @pl.when(cid == 0)
        def _():
            base_e = sid * ept

            cp_s = pltpu.async_copy(src_hbm.at[pl.ds(base_e, ept)], src_v,
                                    sem_in)
            cp_d = pltpu.async_copy(dst_hbm.at[pl.ds(base_e, ept)], dst_v,
                                    sem_in)
            cp_w = pltpu.async_copy(w_hbm.at[pl.ds(base_e, ept)], w_v,
                                    sem_in)
            cp_z = pltpu.async_copy(zeros_hbm.at[pl.ds(sid * z_len, z_len)],
                                    w_sh.at[pl.ds(sid * z_len, z_len)],
                                    sem_z)
            cp_s.wait()
            cp_d.wait()
            cp_w.wait()

            for c in range(n_chunks):
                sl = pl.ds(c * LANES, LANES)
                s16 = src_v[sl]
                d16 = dst_v[sl]
                w16 = w_v[sl]
                m = (s16 < n_in) & (d16 >= base)
                idx = jnp.where(m, s16 * N_OUT + (d16 - base), 0)
                val = jnp.where(m, w16, 0.0)
                out_sl = pl.ds((c % 8) * LANES, LANES)
                idx_v[c // 8, out_sl] = idx
                val_v[c // 8, out_sl] = val

            cp_z.wait()
            plsc.subcore_barrier()

            adds = [
                pltpu.async_copy(val_v.at[j], w_sh.at[idx_v.at[j]], sem_a,
                                 add=True)
                for j in range(n_rows)
            ]
            for cp in adds:
                cp.wait()

            plsc.subcore_barrier()

            pltpu.sync_copy(w_sh.at[pl.ds(sid * z_len, z_len)],
                            out_hbm.at[pl.ds(sid * z_len, z_len)])

    return build_w(src, dst, w, zeros_tab)


def _cast_tc(obs):
    batch, n_in = obs.shape

    def body(obs_ref, out_ref):
        out_ref[...] = obs_ref[...].astype(jnp.bfloat16)

    return pl.pallas_call(
        body,
        grid=(2,),
        in_specs=[pl.BlockSpec((batch // 2, n_in), lambda i: (i, 0))],
        out_specs=pl.BlockSpec((batch // 2, n_in), lambda i: (i, 0)),
        out_shape=jax.ShapeDtypeStruct((batch, n_in), jnp.bfloat16),
    )(obs)


def _matmul_tc(obs, w_table, bias):
    batch, n_in = obs.shape
    n_rec = bias.shape[0]
    base = n_rec - N_OUT
    n_blocks = batch // BATCH_BLOCK

    def body(obs_ref, w_ref, b_ref, out_ref):
        acc = jnp.dot(obs_ref[...], w_ref[...].astype(jnp.bfloat16),
                      preferred_element_type=jnp.float32)
        b = b_ref[0, base:base + N_OUT]
        out_ref[...] = jnp.tanh(acc + b[None, :])

    return pl.pallas_call(
        body,
        grid=(n_blocks,),
        in_specs=[
            pl.BlockSpec((BATCH_BLOCK, n_in), lambda i: (i, 0)),
            pl.BlockSpec((n_in, N_OUT), lambda i: (0, 0)),
            pl.BlockSpec((1, n_rec), lambda i: (0, 0)),
        ],
        out_specs=pl.BlockSpec((BATCH_BLOCK, N_OUT), lambda i: (i, 0)),
        out_shape=jax.ShapeDtypeStruct((batch, N_OUT), jnp.float32),
    )(obs, w_table, bias.reshape(1, n_rec))


def kernel(obs, src, dst, w, bias):
    n_in = obs.shape[1]
    n_rec = bias.shape[0]
    src = src.astype(jnp.int32)
    dst = dst.astype(jnp.int32)
    zeros_tab = jnp.zeros((n_in * N_OUT,), jnp.float32)
    w_flat = _build_w_sc(src, dst, w, zeros_tab, n_in, n_rec)
    obs16 = _cast_tc(obs)
    return _matmul_tc(obs16, w_flat.reshape(n_in, N_OUT), bias)


# speedup vs baseline: 1.3382x; 1.2951x over previous
import functools

import jax
import jax.numpy as jnp
from jax import lax
from jax.experimental import pallas as pl
from jax.experimental.pallas import tpu as pltpu
from jax.experimental.pallas import tpu_sc as plsc

N_OUT = 64
LANES = 16
N_SUB = 16
BATCH_BLOCK = 2048


def _build_w_sc(src, dst, w, zeros_tab, n_in, n_rec):
    E = src.shape[0]
    w_size = n_in * N_OUT
    w_pad = w_size + 128
    base = n_rec - N_OUT
    ept = E // N_SUB
    n_chunks = ept // LANES
    n_rows = ept // 128
    z_len = w_size // N_SUB
    o_len = w_size // N_SUB

    mesh = plsc.VectorSubcoreMesh(core_axis_name="c", subcore_axis_name="s",
                                  num_cores=1)

    @functools.partial(
        pl.kernel,
        mesh=mesh,
        out_type=jax.ShapeDtypeStruct((w_size,), jnp.float32),
        compiler_params=pltpu.CompilerParams(needs_layout_passes=False),
        scratch_types=[
            pltpu.VMEM((ept,), jnp.int32),
            pltpu.VMEM((ept,), jnp.int32),
            pltpu.VMEM((ept,), jnp.float32),
            pltpu.VMEM((n_rows, 128), jnp.int32),
            pltpu.VMEM((n_rows, 128), jnp.float32),
            pltpu.VMEM_SHARED((w_pad,), jnp.float32),
            pltpu.SemaphoreType.DMA,
            pltpu.SemaphoreType.DMA,
            pltpu.SemaphoreType.DMA,
        ],
    )
    def build_w(src_hbm, dst_hbm, w_hbm, zeros_hbm, out_hbm,
                src_v, dst_v, w_v, idx_v, val_v, w_sh,
                sem_in, sem_z, sem_a):
        cid = lax.axis_index("c")
        sid = lax.axis_index("s")

        @pl.when(cid == 0)
        def _():
            base_e = sid * ept

            cp_s = pltpu.async_copy(src_hbm.at[pl.ds(base_e, ept)], src_v,
                                    sem_in)
            cp_d = pltpu.async_copy(dst_hbm.at[pl.ds(base_e, ept)], dst_v,
                                    sem_in)
            cp_w = pltpu.async_copy(w_hbm.at[pl.ds(base_e, ept)], w_v,
                                    sem_in)
            cp_z = pltpu.async_copy(zeros_hbm.at[pl.ds(sid * z_len, z_len)],
                                    w_sh.at[pl.ds(sid * z_len, z_len)],
                                    sem_z)
            cp_s.wait()
            cp_d.wait()
            cp_w.wait()

            for c in range(n_chunks):
                sl = pl.ds(c * LANES, LANES)
                s16 = src_v[sl]
                d16 = dst_v[sl]
                w16 = w_v[sl]
                m = (s16 < n_in) & (d16 >= base)
                lane0 = (c % 8) * LANES
                dump = w_size + lane0 + lax.iota(jnp.int32, 16)
                idx = jnp.where(m, s16 * N_OUT + (d16 - base), dump)
                val = jnp.where(m, w16, 0.0)
                out_sl = pl.ds(lane0, LANES)
                idx_v[c // 8, out_sl] = idx
                val_v[c // 8, out_sl] = val

            cp_z.wait()
            plsc.subcore_barrier()

            adds = [
                pltpu.async_copy(val_v.at[j], w_sh.at[idx_v.at[j]], sem_a,
                                 add=True)
                for j in range(n_rows)
            ]
            for cp in adds:
                cp.wait()

            plsc.subcore_barrier()

            pltpu.sync_copy(w_sh.at[pl.ds(sid * o_len, o_len)],
                            out_hbm.at[pl.ds(sid * o_len, o_len)])

    return build_w(src, dst, w, zeros_tab)


def _matmul_tc(obs, w_table, bias):
    batch, n_in = obs.shape
    n_rec = bias.shape[0]
    base = n_rec - N_OUT
    n_blocks = batch // BATCH_BLOCK

    def body(obs_ref, w_ref, b_ref, out_ref):
        acc = jnp.dot(obs_ref[...], w_ref[...],
                      preferred_element_type=jnp.float32)
        b = b_ref[0, base:base + N_OUT]
        out_ref[...] = jnp.tanh(acc + b[None, :])

    return pl.pallas_call(
        body,
        grid=(n_blocks,),
        in_specs=[
            pl.BlockSpec((BATCH_BLOCK, n_in), lambda i: (i, 0)),
            pl.BlockSpec((n_in, N_OUT), lambda i: (0, 0)),
            pl.BlockSpec((1, n_rec), lambda i: (0, 0)),
        ],
        out_specs=pl.BlockSpec((BATCH_BLOCK, N_OUT), lambda i: (i, 0)),
        out_shape=jax.ShapeDtypeStruct((batch, N_OUT), jnp.float32),
    )(obs, w_table, bias.reshape(1, n_rec))


def kernel(obs, src, dst, w, bias):
    n_in = obs.shape[1]
    n_rec = bias.shape[0]
    src = src.astype(jnp.int32)
    dst = dst.astype(jnp.int32)
    zeros_tab = jnp.zeros((n_in * N_OUT,), jnp.float32)
    w_flat = _build_w_sc(src, dst, w, zeros_tab, n_in, n_rec)
    return _matmul_tc(obs, w_flat.reshape(n_in, N_OUT), bias)
